# baseline (device time: 432179 ns/iter reference)
import functools

import jax
import jax.numpy as jnp
from jax import lax
from jax.experimental import pallas as pl
from jax.experimental.pallas import tpu as pltpu

N_DEV = 16
M = 2048
N = 2048
CHUNK = M // N_DEV
N_STEPS = 2 * (N_DEV - 1)


def _gelu(z):
    return 0.5 * z * (1.0 + jnp.tanh(0.7978845608 * (z + 0.044715 * z * z * z)))


def kernel(A, B):
    m, k_per = A.shape
    _, n = B.shape

    def body(a_ref, b_ref, out_ref, comm_ref, send_sems, recv_sems, credit_sem):
        my = lax.axis_index("i")
        left = jnp.mod(my - 1, N_DEV)
        right = jnp.mod(my + 1, N_DEV)

        barrier_sem = pltpu.get_barrier_semaphore()
        for nbr in (left, right):
            pl.semaphore_signal(
                barrier_sem, inc=1, device_id=(nbr,),
                device_id_type=pl.DeviceIdType.MESH,
            )
        pl.semaphore_wait(barrier_sem, 2)

        out_ref[...] = jnp.dot(
            a_ref[...], b_ref[...], preferred_element_type=jnp.float32
        )

        def row_block(c):
            return pl.ds(c * CHUNK, CHUNK)

        def ring_step(src_c, dst_ref, k):
            p = k % 2
            rdma = pltpu.make_async_remote_copy(
                src_ref=out_ref.at[row_block(src_c), :],
                dst_ref=dst_ref,
                send_sem=send_sems.at[p],
                recv_sem=recv_sems.at[p],
                device_id=(right,),
                device_id_type=pl.DeviceIdType.MESH,
            )
            if k >= 2:
                pl.semaphore_wait(credit_sem, 1)
            rdma.start()
            rdma.wait()
            if k <= N_STEPS - 3:
                pl.semaphore_signal(
                    credit_sem, inc=1, device_id=(left,),
                    device_id_type=pl.DeviceIdType.MESH,
                )

        for s in range(N_DEV - 1):
            src_c = jnp.mod(my - s, N_DEV)
            ring_step(src_c, comm_ref.at[s % 2], s)
            acc_c = jnp.mod(my - s - 1, N_DEV)
            out_ref[row_block(acc_c), :] = (
                out_ref[row_block(acc_c), :] + comm_ref[s % 2]
            )

        own_c = jnp.mod(my + 1, N_DEV)
        out_ref[row_block(own_c), :] = _gelu(out_ref[row_block(own_c), :])

        for t in range(N_DEV - 1):
            src_c = jnp.mod(my + 1 - t, N_DEV)
            ring_step(src_c, out_ref.at[row_block(src_c), :], (N_DEV - 1) + t)

        @functools.partial(pl.run_scoped, sem=pltpu.SemaphoreType.REGULAR)
        def _(sem):
            for nbr in (left, right):
                pl.semaphore_signal(
                    sem, inc=1, device_id=(nbr,),
                    device_id_type=pl.DeviceIdType.MESH,
                )
            pl.semaphore_wait(sem, 2)

    return pl.pallas_call(
        body,
        out_shape=jax.ShapeDtypeStruct((m, n), jnp.float32),
        in_specs=[
            pl.BlockSpec(memory_space=pltpu.VMEM),
            pl.BlockSpec(memory_space=pltpu.VMEM),
        ],
        out_specs=pl.BlockSpec(memory_space=pltpu.VMEM),
        scratch_shapes=[
            pltpu.VMEM((2, CHUNK, N), jnp.float32),
            pltpu.SemaphoreType.DMA((2,)),
            pltpu.SemaphoreType.DMA((2,)),
            pltpu.SemaphoreType.REGULAR,
        ],
        compiler_params=pltpu.CompilerParams(collective_id=0),
    )(A, B)


# device time: 304636 ns/iter; 1.4187x vs baseline; 1.4187x over previous
import functools

import jax
import jax.numpy as jnp
from jax import lax
from jax.experimental import pallas as pl
from jax.experimental.pallas import tpu as pltpu

N_DEV = 16
M = 2048
N = 2048
CHUNK = M // N_DEV
HALF = CHUNK // 2
N_STEPS = 2 * (N_DEV - 1)


def _gelu(z):
    return 0.5 * z * (1.0 + jnp.tanh(0.7978845608 * (z + 0.044715 * z * z * z)))


def kernel(A, B):
    m, k_per = A.shape
    _, n = B.shape

    def body(
        a_ref, b_ref, out_ref,
        comm_p, comm_m,
        send_p, recv_p, send_m, recv_m,
        credit_p, credit_m,
    ):
        my = lax.axis_index("i")
        left = jnp.mod(my - 1, N_DEV)
        right = jnp.mod(my + 1, N_DEV)

        barrier_sem = pltpu.get_barrier_semaphore()
        for nbr in (left, right):
            pl.semaphore_signal(
                barrier_sem, inc=1, device_id=(nbr,),
                device_id_type=pl.DeviceIdType.MESH,
            )
        pl.semaphore_wait(barrier_sem, 2)

        out_ref[...] = jnp.dot(
            a_ref[...], b_ref[...], preferred_element_type=jnp.float32
        )

        def top(c):
            return pl.ds(c * CHUNK, HALF)

        def bot(c):
            return pl.ds(c * CHUNK + HALF, HALF)

        def ring_step(src_p_c, dst_p_ref, src_m_c, dst_m_ref, k):
            p = k % 2
            rdma_p = pltpu.make_async_remote_copy(
                src_ref=out_ref.at[top(src_p_c), :],
                dst_ref=dst_p_ref,
                send_sem=send_p.at[p],
                recv_sem=recv_p.at[p],
                device_id=(right,),
                device_id_type=pl.DeviceIdType.MESH,
            )
            rdma_m = pltpu.make_async_remote_copy(
                src_ref=out_ref.at[bot(src_m_c), :],
                dst_ref=dst_m_ref,
                send_sem=send_m.at[p],
                recv_sem=recv_m.at[p],
                device_id=(left,),
                device_id_type=pl.DeviceIdType.MESH,
            )
            if k >= 2:
                pl.semaphore_wait(credit_p, 1)
                pl.semaphore_wait(credit_m, 1)
            rdma_p.start()
            rdma_m.start()
            rdma_p.wait()
            rdma_m.wait()
            if k <= N_STEPS - 3:
                pl.semaphore_signal(
                    credit_p, inc=1, device_id=(left,),
                    device_id_type=pl.DeviceIdType.MESH,
                )
                pl.semaphore_signal(
                    credit_m, inc=1, device_id=(right,),
                    device_id_type=pl.DeviceIdType.MESH,
                )

        for s in range(N_DEV - 1):
            sp = jnp.mod(my - s, N_DEV)
            sm = jnp.mod(my + s, N_DEV)
            ring_step(sp, comm_p.at[s % 2], sm, comm_m.at[s % 2], s)
            ap = jnp.mod(my - s - 1, N_DEV)
            am = jnp.mod(my + s + 1, N_DEV)
            out_ref[top(ap), :] = out_ref[top(ap), :] + comm_p[s % 2]
            out_ref[bot(am), :] = out_ref[bot(am), :] + comm_m[s % 2]

        own_p = jnp.mod(my + 1, N_DEV)
        own_m = jnp.mod(my - 1, N_DEV)
        out_ref[top(own_p), :] = _gelu(out_ref[top(own_p), :])
        out_ref[bot(own_m), :] = _gelu(out_ref[bot(own_m), :])

        for t in range(N_DEV - 1):
            sp = jnp.mod(my + 1 - t, N_DEV)
            sm = jnp.mod(my - 1 + t, N_DEV)
            ring_step(
                sp, out_ref.at[top(sp), :],
                sm, out_ref.at[bot(sm), :],
                (N_DEV - 1) + t,
            )

        @functools.partial(pl.run_scoped, sem=pltpu.SemaphoreType.REGULAR)
        def _(sem):
            for nbr in (left, right):
                pl.semaphore_signal(
                    sem, inc=1, device_id=(nbr,),
                    device_id_type=pl.DeviceIdType.MESH,
                )
            pl.semaphore_wait(sem, 2)

    return pl.pallas_call(
        body,
        out_shape=jax.ShapeDtypeStruct((m, n), jnp.float32),
        in_specs=[
            pl.BlockSpec(memory_space=pltpu.VMEM),
            pl.BlockSpec(memory_space=pltpu.VMEM),
        ],
        out_specs=pl.BlockSpec(memory_space=pltpu.VMEM),
        scratch_shapes=[
            pltpu.VMEM((2, HALF, N), jnp.float32),
            pltpu.VMEM((2, HALF, N), jnp.float32),
            pltpu.SemaphoreType.DMA((2,)),
            pltpu.SemaphoreType.DMA((2,)),
            pltpu.SemaphoreType.DMA((2,)),
            pltpu.SemaphoreType.DMA((2,)),
            pltpu.SemaphoreType.REGULAR,
            pltpu.SemaphoreType.REGULAR,
        ],
        compiler_params=pltpu.CompilerParams(collective_id=0),
    )(A, B)


# device time: 210952 ns/iter; 2.0487x vs baseline; 1.4441x over previous
import functools

import jax
import jax.numpy as jnp
from jax import lax
from jax.experimental import pallas as pl
from jax.experimental.pallas import tpu as pltpu

N_DEV = 16
M = 2048
N = 2048
CHUNK = M // N_DEV
HALF = CHUNK // 2
Q = 2
QROWS = HALF // Q
NSLOT = 4
RS = N_DEV - 1
NMSG = 2 * RS * Q


def _gelu(z):
    return 0.5 * z * (1.0 + jnp.tanh(0.7978845608 * (z + 0.044715 * z * z * z)))


def kernel(A, B):
    m, k_per = A.shape
    _, n = B.shape

    def body(
        a_ref, b_ref, out_ref,
        buf_p, buf_m,
        send_p, recv_p, send_m, recv_m,
        credit_p, credit_m,
    ):
        my = lax.axis_index("i")
        left = jnp.mod(my - 1, N_DEV)
        right = jnp.mod(my + 1, N_DEV)

        barrier_sem = pltpu.get_barrier_semaphore()
        for nbr in (left, right):
            pl.semaphore_signal(
                barrier_sem, inc=1, device_id=(nbr,),
                device_id_type=pl.DeviceIdType.MESH,
            )
        pl.semaphore_wait(barrier_sem, 2)

        def tq(c, q):
            return pl.ds(c * CHUNK + q * QROWS, QROWS)

        def bq(c, q):
            return pl.ds(c * CHUNK + HALF + q * QROWS, QROWS)

        def dot_half(r0):
            out_ref[pl.ds(r0, HALF), :] = jnp.dot(
                a_ref[pl.ds(r0, HALF), :], b_ref[...],
                preferred_element_type=jnp.float32,
            )

        sems = {
            "p": (send_p, recv_p, credit_p, right, left),
            "m": (send_m, recv_m, credit_m, left, right),
        }
        pend = {"p": {}, "m": {}}

        def issue(d, mm, src_slice, dst_ref):
            send_sems, recv_sems, credit, tgt, _src_nbr = sems[d]
            desc = pltpu.make_async_remote_copy(
                src_ref=out_ref.at[src_slice, :],
                dst_ref=dst_ref,
                send_sem=send_sems.at[mm % NSLOT],
                recv_sem=recv_sems.at[mm % NSLOT],
                device_id=(tgt,),
                device_id_type=pl.DeviceIdType.MESH,
            )
            if mm >= NSLOT:
                pend[d][mm - NSLOT].wait_send()
                pl.semaphore_wait(credit, 1)
            desc.start()
            pend[d][mm] = desc

        def wait_msg(d, mm):
            pend[d][mm].wait_recv()

        def credit_back(d, mm):
            if mm < NMSG - NSLOT:
                _s, _r, credit, _tgt, src_nbr = sems[d]
                pl.semaphore_signal(
                    credit, inc=1, device_id=(src_nbr,),
                    device_id_type=pl.DeviceIdType.MESH,
                )

        dot_half(my * CHUNK)
        dot_half(my * CHUNK + HALF)
        for q in range(Q):
            issue("p", q, tq(my, q), buf_p.at[q])
            issue("m", q, bq(my, q), buf_m.at[q])

        for s in range(RS):
            cp = jnp.mod(my - s - 1, N_DEV)
            cm = jnp.mod(my + s + 1, N_DEV)
            dot_half(cp * CHUNK)
            dot_half(cm * CHUNK + HALF)
            for q in range(Q):
                mm = s * Q + q
                wait_msg("p", mm)
                out_ref[tq(cp, q), :] = out_ref[tq(cp, q), :] + buf_p[mm % NSLOT]
                credit_back("p", mm)
                if s < RS - 1:
                    issue("p", mm + Q, tq(cp, q), buf_p.at[(mm + Q) % NSLOT])
                wait_msg("m", mm)
                out_ref[bq(cm, q), :] = out_ref[bq(cm, q), :] + buf_m[mm % NSLOT]
                credit_back("m", mm)
                if s < RS - 1:
                    issue("m", mm + Q, bq(cm, q), buf_m.at[(mm + Q) % NSLOT])

        own_p = jnp.mod(my + 1, N_DEV)
        own_m = jnp.mod(my - 1, N_DEV)
        out_ref[pl.ds(own_p * CHUNK, HALF), :] = _gelu(
            out_ref[pl.ds(own_p * CHUNK, HALF), :]
        )
        out_ref[pl.ds(own_m * CHUNK + HALF, HALF), :] = _gelu(
            out_ref[pl.ds(own_m * CHUNK + HALF, HALF), :]
        )

        for q in range(Q):
            m0 = RS * Q + q
            issue("p", m0, tq(own_p, q), out_ref.at[tq(own_p, q), :])
            issue("m", m0, bq(own_m, q), out_ref.at[bq(own_m, q), :])
        for t in range(RS):
            rp = jnp.mod(my - t, N_DEV)
            rm = jnp.mod(my + t, N_DEV)
            for q in range(Q):
                mm = (RS + t) * Q + q
                wait_msg("p", mm)
                credit_back("p", mm)
                if t < RS - 1:
                    issue("p", mm + Q, tq(rp, q), out_ref.at[tq(rp, q), :])
                wait_msg("m", mm)
                credit_back("m", mm)
                if t < RS - 1:
                    issue("m", mm + Q, bq(rm, q), out_ref.at[bq(rm, q), :])

        for mm in range(NMSG - NSLOT, NMSG):
            pend["p"][mm].wait_send()
            pend["m"][mm].wait_send()

        @functools.partial(pl.run_scoped, sem=pltpu.SemaphoreType.REGULAR)
        def _(sem):
            for nbr in (left, right):
                pl.semaphore_signal(
                    sem, inc=1, device_id=(nbr,),
                    device_id_type=pl.DeviceIdType.MESH,
                )
            pl.semaphore_wait(sem, 2)

    return pl.pallas_call(
        body,
        out_shape=jax.ShapeDtypeStruct((m, n), jnp.float32),
        in_specs=[
            pl.BlockSpec(memory_space=pltpu.VMEM),
            pl.BlockSpec(memory_space=pltpu.VMEM),
        ],
        out_specs=pl.BlockSpec(memory_space=pltpu.VMEM),
        scratch_shapes=[
            pltpu.VMEM((NSLOT, QROWS, N), jnp.float32),
            pltpu.VMEM((NSLOT, QROWS, N), jnp.float32),
            pltpu.SemaphoreType.DMA((NSLOT,)),
            pltpu.SemaphoreType.DMA((NSLOT,)),
            pltpu.SemaphoreType.DMA((NSLOT,)),
            pltpu.SemaphoreType.DMA((NSLOT,)),
            pltpu.SemaphoreType.REGULAR,
            pltpu.SemaphoreType.REGULAR,
        ],
        compiler_params=pltpu.CompilerParams(collective_id=0),
    )(A, B)


# device time: 205187 ns/iter; 2.1063x vs baseline; 1.0281x over previous
import functools

import jax
import jax.numpy as jnp
from jax import lax
from jax.experimental import pallas as pl
from jax.experimental.pallas import tpu as pltpu

N_DEV = 16
M = 2048
N = 2048

RING = (0, 4, 8, 12, 15, 11, 7, 3, 2, 6, 10, 14, 13, 9, 5, 1)
CHUNK = M // N_DEV
HALF = CHUNK // 2
Q = 2
QROWS = HALF // Q
NSLOT = 4
RS = N_DEV - 1
NMSG = 2 * RS * Q


def _gelu(z):
    return 0.5 * z * (1.0 + jnp.tanh(0.7978845608 * (z + 0.044715 * z * z * z)))


def kernel(A, B):
    m, k_per = A.shape
    _, n = B.shape

    def body(
        a_ref, b_ref, out_ref,
        buf_p, buf_m,
        send_p, recv_p, send_m, recv_m,
        credit_p, credit_m,
    ):
        my_log = lax.axis_index("i")
        my = jnp.int32(0)
        right = jnp.int32(0)
        left = jnp.int32(0)
        for j in range(N_DEV):
            here = my_log == RING[j]
            my = jnp.where(here, j, my)
            right = jnp.where(here, RING[(j + 1) % N_DEV], right)
            left = jnp.where(here, RING[(j - 1) % N_DEV], left)

        barrier_sem = pltpu.get_barrier_semaphore()
        for nbr in (left, right):
            pl.semaphore_signal(
                barrier_sem, inc=1, device_id=(nbr,),
                device_id_type=pl.DeviceIdType.MESH,
            )
        pl.semaphore_wait(barrier_sem, 2)

        def tq(c, q):
            return pl.ds(c * CHUNK + q * QROWS, QROWS)

        def bq(c, q):
            return pl.ds(c * CHUNK + HALF + q * QROWS, QROWS)

        def dot_half(r0):
            out_ref[pl.ds(r0, HALF), :] = jnp.dot(
                a_ref[pl.ds(r0, HALF), :], b_ref[...],
                preferred_element_type=jnp.float32,
            )

        sems = {
            "p": (send_p, recv_p, credit_p, right, left),
            "m": (send_m, recv_m, credit_m, left, right),
        }
        pend = {"p": {}, "m": {}}

        def issue(d, mm, src_slice, dst_ref):
            send_sems, recv_sems, credit, tgt, _src_nbr = sems[d]
            desc = pltpu.make_async_remote_copy(
                src_ref=out_ref.at[src_slice, :],
                dst_ref=dst_ref,
                send_sem=send_sems.at[mm % NSLOT],
                recv_sem=recv_sems.at[mm % NSLOT],
                device_id=(tgt,),
                device_id_type=pl.DeviceIdType.MESH,
            )
            if mm >= NSLOT:
                pend[d][mm - NSLOT].wait_send()
                pl.semaphore_wait(credit, 1)
            desc.start()
            pend[d][mm] = desc

        def wait_msg(d, mm):
            pend[d][mm].wait_recv()

        def credit_back(d, mm):
            if mm < NMSG - NSLOT:
                _s, _r, credit, _tgt, src_nbr = sems[d]
                pl.semaphore_signal(
                    credit, inc=1, device_id=(src_nbr,),
                    device_id_type=pl.DeviceIdType.MESH,
                )

        dot_half(my * CHUNK)
        dot_half(my * CHUNK + HALF)
        for q in range(Q):
            issue("p", q, tq(my, q), buf_p.at[q])
            issue("m", q, bq(my, q), buf_m.at[q])

        for s in range(RS):
            cp = jnp.mod(my - s - 1, N_DEV)
            cm = jnp.mod(my + s + 1, N_DEV)
            dot_half(cp * CHUNK)
            dot_half(cm * CHUNK + HALF)
            for q in range(Q):
                mm = s * Q + q
                wait_msg("p", mm)
                out_ref[tq(cp, q), :] = out_ref[tq(cp, q), :] + buf_p[mm % NSLOT]
                credit_back("p", mm)
                if s < RS - 1:
                    issue("p", mm + Q, tq(cp, q), buf_p.at[(mm + Q) % NSLOT])
                wait_msg("m", mm)
                out_ref[bq(cm, q), :] = out_ref[bq(cm, q), :] + buf_m[mm % NSLOT]
                credit_back("m", mm)
                if s < RS - 1:
                    issue("m", mm + Q, bq(cm, q), buf_m.at[(mm + Q) % NSLOT])

        own_p = jnp.mod(my + 1, N_DEV)
        own_m = jnp.mod(my - 1, N_DEV)
        out_ref[pl.ds(own_p * CHUNK, HALF), :] = _gelu(
            out_ref[pl.ds(own_p * CHUNK, HALF), :]
        )
        out_ref[pl.ds(own_m * CHUNK + HALF, HALF), :] = _gelu(
            out_ref[pl.ds(own_m * CHUNK + HALF, HALF), :]
        )

        for q in range(Q):
            m0 = RS * Q + q
            issue("p", m0, tq(own_p, q), out_ref.at[tq(own_p, q), :])
            issue("m", m0, bq(own_m, q), out_ref.at[bq(own_m, q), :])
        for t in range(RS):
            rp = jnp.mod(my - t, N_DEV)
            rm = jnp.mod(my + t, N_DEV)
            for q in range(Q):
                mm = (RS + t) * Q + q
                wait_msg("p", mm)
                credit_back("p", mm)
                if t < RS - 1:
                    issue("p", mm + Q, tq(rp, q), out_ref.at[tq(rp, q), :])
                wait_msg("m", mm)
                credit_back("m", mm)
                if t < RS - 1:
                    issue("m", mm + Q, bq(rm, q), out_ref.at[bq(rm, q), :])

        for mm in range(NMSG - NSLOT, NMSG):
            pend["p"][mm].wait_send()
            pend["m"][mm].wait_send()

        @functools.partial(pl.run_scoped, sem=pltpu.SemaphoreType.REGULAR)
        def _(sem):
            for nbr in (left, right):
                pl.semaphore_signal(
                    sem, inc=1, device_id=(nbr,),
                    device_id_type=pl.DeviceIdType.MESH,
                )
            pl.semaphore_wait(sem, 2)

    return pl.pallas_call(
        body,
        out_shape=jax.ShapeDtypeStruct((m, n), jnp.float32),
        in_specs=[
            pl.BlockSpec(memory_space=pltpu.VMEM),
            pl.BlockSpec(memory_space=pltpu.VMEM),
        ],
        out_specs=pl.BlockSpec(memory_space=pltpu.VMEM),
        scratch_shapes=[
            pltpu.VMEM((NSLOT, QROWS, N), jnp.float32),
            pltpu.VMEM((NSLOT, QROWS, N), jnp.float32),
            pltpu.SemaphoreType.DMA((NSLOT,)),
            pltpu.SemaphoreType.DMA((NSLOT,)),
            pltpu.SemaphoreType.DMA((NSLOT,)),
            pltpu.SemaphoreType.DMA((NSLOT,)),
            pltpu.SemaphoreType.REGULAR,
            pltpu.SemaphoreType.REGULAR,
        ],
        compiler_params=pltpu.CompilerParams(collective_id=0),
    )(A, B)


# device time: 126289 ns/iter; 3.4221x vs baseline; 1.6247x over previous
import functools

import jax
import jax.numpy as jnp
from jax import lax
from jax.experimental import pallas as pl
from jax.experimental.pallas import tpu as pltpu

N_DEV = 16
M = 2048
N = 2048
CHUNK = M // N_DEV
HALF = CHUNK // 2
Q = 4
QROWS = HALF // Q
NSLOT = 8
RS = N_DEV - 1
NMSG = 2 * RS * Q

RING = (0, 4, 8, 12, 15, 11, 7, 3, 2, 6, 10, 14, 13, 9, 5, 1)


def _gelu(z):
    return 0.5 * z * (1.0 + jnp.tanh(0.7978845608 * (z + 0.044715 * z * z * z)))


def kernel(A, B):
    m, k_per = A.shape
    _, n = B.shape

    def body(
        a_ref, b_ref, out_ref,
        sbuf_p, rbuf_p, sbuf_m, rbuf_m,
        send_p, recv_p, send_m, recv_m,
        credit_p, credit_m,
    ):
        my_log = lax.axis_index("i")
        my = jnp.int32(0)
        right = jnp.int32(0)
        left = jnp.int32(0)
        for j in range(N_DEV):
            here = my_log == RING[j]
            my = jnp.where(here, j, my)
            right = jnp.where(here, RING[(j + 1) % N_DEV], right)
            left = jnp.where(here, RING[(j - 1) % N_DEV], left)

        barrier_sem = pltpu.get_barrier_semaphore()
        for nbr in (left, right):
            pl.semaphore_signal(
                barrier_sem, inc=1, device_id=(nbr,),
                device_id_type=pl.DeviceIdType.MESH,
            )
        pl.semaphore_wait(barrier_sem, 2)

        def tq(c, q):
            return pl.ds(c * CHUNK + q * QROWS, QROWS)

        def bq(c, q):
            return pl.ds(c * CHUNK + HALF + q * QROWS, QROWS)

        def dot_half(r0):
            out_ref[pl.ds(r0, HALF), :] = jnp.dot(
                a_ref[pl.ds(r0, HALF), :], b_ref[...],
                preferred_element_type=jnp.float32,
            )

        sems = {
            "p": (send_p, recv_p, credit_p, sbuf_p, rbuf_p, right, left),
            "m": (send_m, recv_m, credit_m, sbuf_m, rbuf_m, left, right),
        }
        pend = {"p": {}, "m": {}}

        def issue(d, mm, payload):
            send_sems, recv_sems, credit, sbuf, rbuf, tgt, _ = sems[d]
            slot = mm % NSLOT
            if mm >= NSLOT:
                pend[d][mm - NSLOT].wait_send()
                pl.semaphore_wait(credit, 1)
            sbuf[slot] = payload
            desc = pltpu.make_async_remote_copy(
                src_ref=sbuf.at[slot],
                dst_ref=rbuf.at[slot],
                send_sem=send_sems.at[slot],
                recv_sem=recv_sems.at[slot],
                device_id=(tgt,),
                device_id_type=pl.DeviceIdType.MESH,
            )
            desc.start()
            pend[d][mm] = desc

        def credit_back(d, mm):
            if mm < NMSG - NSLOT:
                _s, _r, credit, _sb, _rb, _tgt, src_nbr = sems[d]
                pl.semaphore_signal(
                    credit, inc=1, device_id=(src_nbr,),
                    device_id_type=pl.DeviceIdType.MESH,
                )

        dot_half(my * CHUNK)
        dot_half(my * CHUNK + HALF)
        for q in range(Q):
            issue("p", q, out_ref[tq(my, q), :].astype(jnp.bfloat16))
            issue("m", q, out_ref[bq(my, q), :].astype(jnp.bfloat16))

        for s in range(RS):
            cp = jnp.mod(my - s - 1, N_DEV)
            cm = jnp.mod(my + s + 1, N_DEV)
            dot_half(cp * CHUNK)
            dot_half(cm * CHUNK + HALF)
            for q in range(Q):
                mm = s * Q + q
                pend["p"][mm].wait_recv()
                out_ref[tq(cp, q), :] = (
                    out_ref[tq(cp, q), :]
                    + rbuf_p[mm % NSLOT].astype(jnp.float32)
                )
                credit_back("p", mm)
                if s < RS - 1:
                    issue("p", mm + Q, out_ref[tq(cp, q), :].astype(jnp.bfloat16))
                pend["m"][mm].wait_recv()
                out_ref[bq(cm, q), :] = (
                    out_ref[bq(cm, q), :]
                    + rbuf_m[mm % NSLOT].astype(jnp.float32)
                )
                credit_back("m", mm)
                if s < RS - 1:
                    issue("m", mm + Q, out_ref[bq(cm, q), :].astype(jnp.bfloat16))

        own_p = jnp.mod(my + 1, N_DEV)
        own_m = jnp.mod(my - 1, N_DEV)
        out_ref[pl.ds(own_p * CHUNK, HALF), :] = _gelu(
            out_ref[pl.ds(own_p * CHUNK, HALF), :]
        )
        out_ref[pl.ds(own_m * CHUNK + HALF, HALF), :] = _gelu(
            out_ref[pl.ds(own_m * CHUNK + HALF, HALF), :]
        )

        for q in range(Q):
            m0 = RS * Q + q
            issue("p", m0, out_ref[tq(own_p, q), :].astype(jnp.bfloat16))
            issue("m", m0, out_ref[bq(own_m, q), :].astype(jnp.bfloat16))
        for t in range(RS):
            rp = jnp.mod(my - t, N_DEV)
            rm = jnp.mod(my + t, N_DEV)
            for q in range(Q):
                mm = (RS + t) * Q + q
                pend["p"][mm].wait_recv()
                if t < RS - 1:
                    issue("p", mm + Q, rbuf_p[mm % NSLOT])
                out_ref[tq(rp, q), :] = rbuf_p[mm % NSLOT].astype(jnp.float32)
                credit_back("p", mm)
                pend["m"][mm].wait_recv()
                if t < RS - 1:
                    issue("m", mm + Q, rbuf_m[mm % NSLOT])
                out_ref[bq(rm, q), :] = rbuf_m[mm % NSLOT].astype(jnp.float32)
                credit_back("m", mm)

        for mm in range(NMSG - NSLOT, NMSG):
            pend["p"][mm].wait_send()
            pend["m"][mm].wait_send()

        @functools.partial(pl.run_scoped, sem=pltpu.SemaphoreType.REGULAR)
        def _(sem):
            for nbr in (left, right):
                pl.semaphore_signal(
                    sem, inc=1, device_id=(nbr,),
                    device_id_type=pl.DeviceIdType.MESH,
                )
            pl.semaphore_wait(sem, 2)

    return pl.pallas_call(
        body,
        out_shape=jax.ShapeDtypeStruct((m, n), jnp.float32),
        in_specs=[
            pl.BlockSpec(memory_space=pltpu.VMEM),
            pl.BlockSpec(memory_space=pltpu.VMEM),
        ],
        out_specs=pl.BlockSpec(memory_space=pltpu.VMEM),
        scratch_shapes=[
            pltpu.VMEM((NSLOT, QROWS, N), jnp.bfloat16),
            pltpu.VMEM((NSLOT, QROWS, N), jnp.bfloat16),
            pltpu.VMEM((NSLOT, QROWS, N), jnp.bfloat16),
            pltpu.VMEM((NSLOT, QROWS, N), jnp.bfloat16),
            pltpu.SemaphoreType.DMA((NSLOT,)),
            pltpu.SemaphoreType.DMA((NSLOT,)),
            pltpu.SemaphoreType.DMA((NSLOT,)),
            pltpu.SemaphoreType.DMA((NSLOT,)),
            pltpu.SemaphoreType.REGULAR,
            pltpu.SemaphoreType.REGULAR,
        ],
        compiler_params=pltpu.CompilerParams(collective_id=0),
    )(A, B)


# device time: 125722 ns/iter; 3.4376x vs baseline; 1.0045x over previous
import functools

import jax
import jax.numpy as jnp
from jax import lax
from jax.experimental import pallas as pl
from jax.experimental.pallas import tpu as pltpu

N_DEV = 16
M = 2048
N = 2048
CHUNK = M // N_DEV
HALF = CHUNK // 2
Q = 4
QROWS = HALF // Q
NSLOT = 8
RS = N_DEV - 1
NMSG = 2 * RS * Q

RING = (0, 4, 8, 12, 15, 11, 7, 3, 2, 6, 10, 14, 13, 9, 5, 1)


def _gelu(z):
    return 0.5 * z * (1.0 + jnp.tanh(0.7978845608 * (z + 0.044715 * z * z * z)))


def kernel(A, B):
    m, k_per = A.shape
    _, n = B.shape

    def body(
        a_ref, b_ref, out_ref,
        b16_ref,
        sbuf_p, rbuf_p, sbuf_m, rbuf_m,
        send_p, recv_p, send_m, recv_m,
        credit_p, credit_m,
    ):
        my_log = lax.axis_index("i")
        my = jnp.int32(0)
        right = jnp.int32(0)
        left = jnp.int32(0)
        for j in range(N_DEV):
            here = my_log == RING[j]
            my = jnp.where(here, j, my)
            right = jnp.where(here, RING[(j + 1) % N_DEV], right)
            left = jnp.where(here, RING[(j - 1) % N_DEV], left)

        barrier_sem = pltpu.get_barrier_semaphore()
        for nbr in (left, right):
            pl.semaphore_signal(
                barrier_sem, inc=1, device_id=(nbr,),
                device_id_type=pl.DeviceIdType.MESH,
            )
        pl.semaphore_wait(barrier_sem, 2)

        def tq(c, q):
            return pl.ds(c * CHUNK + q * QROWS, QROWS)

        def bq(c, q):
            return pl.ds(c * CHUNK + HALF + q * QROWS, QROWS)

        b16_ref[...] = b_ref[...].astype(jnp.bfloat16)

        def dot_half(r0):
            out_ref[pl.ds(r0, HALF), :] = jnp.dot(
                a_ref[pl.ds(r0, HALF), :].astype(jnp.bfloat16), b16_ref[...],
                preferred_element_type=jnp.float32,
            )

        sems = {
            "p": (send_p, recv_p, credit_p, sbuf_p, rbuf_p, right, left),
            "m": (send_m, recv_m, credit_m, sbuf_m, rbuf_m, left, right),
        }
        pend = {"p": {}, "m": {}}

        def issue(d, mm, payload):
            send_sems, recv_sems, credit, sbuf, rbuf, tgt, _ = sems[d]
            slot = mm % NSLOT
            if mm >= NSLOT:
                pend[d][mm - NSLOT].wait_send()
                pl.semaphore_wait(credit, 1)
            sbuf[slot] = payload
            desc = pltpu.make_async_remote_copy(
                src_ref=sbuf.at[slot],
                dst_ref=rbuf.at[slot],
                send_sem=send_sems.at[slot],
                recv_sem=recv_sems.at[slot],
                device_id=(tgt,),
                device_id_type=pl.DeviceIdType.MESH,
            )
            desc.start()
            pend[d][mm] = desc

        def credit_back(d, mm):
            if mm < NMSG - NSLOT:
                _s, _r, credit, _sb, _rb, _tgt, src_nbr = sems[d]
                pl.semaphore_signal(
                    credit, inc=1, device_id=(src_nbr,),
                    device_id_type=pl.DeviceIdType.MESH,
                )

        dot_half(my * CHUNK)
        dot_half(my * CHUNK + HALF)
        for q in range(Q):
            issue("p", q, out_ref[tq(my, q), :].astype(jnp.bfloat16))
            issue("m", q, out_ref[bq(my, q), :].astype(jnp.bfloat16))

        for s in range(RS):
            cp = jnp.mod(my - s - 1, N_DEV)
            cm = jnp.mod(my + s + 1, N_DEV)
            dot_half(cp * CHUNK)
            dot_half(cm * CHUNK + HALF)
            for q in range(Q):
                mm = s * Q + q
                pend["p"][mm].wait_recv()
                if s < RS - 1:
                    issue(
                        "p", mm + Q,
                        (
                            out_ref[tq(cp, q), :]
                            + rbuf_p[mm % NSLOT].astype(jnp.float32)
                        ).astype(jnp.bfloat16),
                    )
                    credit_back("p", mm)
                else:
                    out_ref[tq(cp, q), :] = (
                        out_ref[tq(cp, q), :]
                        + rbuf_p[mm % NSLOT].astype(jnp.float32)
                    )
                    credit_back("p", mm)
                pend["m"][mm].wait_recv()
                if s < RS - 1:
                    issue(
                        "m", mm + Q,
                        (
                            out_ref[bq(cm, q), :]
                            + rbuf_m[mm % NSLOT].astype(jnp.float32)
                        ).astype(jnp.bfloat16),
                    )
                    credit_back("m", mm)
                else:
                    out_ref[bq(cm, q), :] = (
                        out_ref[bq(cm, q), :]
                        + rbuf_m[mm % NSLOT].astype(jnp.float32)
                    )
                    credit_back("m", mm)

        own_p = jnp.mod(my + 1, N_DEV)
        own_m = jnp.mod(my - 1, N_DEV)
        out_ref[pl.ds(own_p * CHUNK, HALF), :] = _gelu(
            out_ref[pl.ds(own_p * CHUNK, HALF), :]
        )
        out_ref[pl.ds(own_m * CHUNK + HALF, HALF), :] = _gelu(
            out_ref[pl.ds(own_m * CHUNK + HALF, HALF), :]
        )

        for q in range(Q):
            m0 = RS * Q + q
            issue("p", m0, out_ref[tq(own_p, q), :].astype(jnp.bfloat16))
            issue("m", m0, out_ref[bq(own_m, q), :].astype(jnp.bfloat16))
        for t in range(RS):
            rp = jnp.mod(my - t, N_DEV)
            rm = jnp.mod(my + t, N_DEV)
            for q in range(Q):
                mm = (RS + t) * Q + q
                pend["p"][mm].wait_recv()
                if t < RS - 1:
                    issue("p", mm + Q, rbuf_p[mm % NSLOT])
                out_ref[tq(rp, q), :] = rbuf_p[mm % NSLOT].astype(jnp.float32)
                credit_back("p", mm)
                pend["m"][mm].wait_recv()
                if t < RS - 1:
                    issue("m", mm + Q, rbuf_m[mm % NSLOT])
                out_ref[bq(rm, q), :] = rbuf_m[mm % NSLOT].astype(jnp.float32)
                credit_back("m", mm)

        for mm in range(NMSG - NSLOT, NMSG):
            pend["p"][mm].wait_send()
            pend["m"][mm].wait_send()

        @functools.partial(pl.run_scoped, sem=pltpu.SemaphoreType.REGULAR)
        def _(sem):
            for nbr in (left, right):
                pl.semaphore_signal(
                    sem, inc=1, device_id=(nbr,),
                    device_id_type=pl.DeviceIdType.MESH,
                )
            pl.semaphore_wait(sem, 2)

    return pl.pallas_call(
        body,
        out_shape=jax.ShapeDtypeStruct((m, n), jnp.float32),
        in_specs=[
            pl.BlockSpec(memory_space=pltpu.VMEM),
            pl.BlockSpec(memory_space=pltpu.VMEM),
        ],
        out_specs=pl.BlockSpec(memory_space=pltpu.VMEM),
        scratch_shapes=[
            pltpu.VMEM((k_per, n), jnp.bfloat16),
            pltpu.VMEM((NSLOT, QROWS, N), jnp.bfloat16),
            pltpu.VMEM((NSLOT, QROWS, N), jnp.bfloat16),
            pltpu.VMEM((NSLOT, QROWS, N), jnp.bfloat16),
            pltpu.VMEM((NSLOT, QROWS, N), jnp.bfloat16),
            pltpu.SemaphoreType.DMA((NSLOT,)),
            pltpu.SemaphoreType.DMA((NSLOT,)),
            pltpu.SemaphoreType.DMA((NSLOT,)),
            pltpu.SemaphoreType.DMA((NSLOT,)),
            pltpu.SemaphoreType.REGULAR,
            pltpu.SemaphoreType.REGULAR,
        ],
        compiler_params=pltpu.CompilerParams(collective_id=0),
    )(A, B)


# device time: 117720 ns/iter; 3.6712x vs baseline; 1.0680x over previous
import functools

import jax
import jax.numpy as jnp
from jax import lax
from jax.experimental import pallas as pl
from jax.experimental.pallas import tpu as pltpu

N_DEV = 16
M = 2048
N = 2048
CHUNK = M // N_DEV
HALF = CHUNK // 2
Q = 4
QROWS = HALF // Q
NSLOT = 8
OSLOT = 8
RS = N_DEV - 1
NMSG = 2 * RS * Q

RING = (0, 4, 8, 12, 15, 11, 7, 3, 2, 6, 10, 14, 13, 9, 5, 1)


def _gelu(z):
    return 0.5 * z * (1.0 + jnp.tanh(0.7978845608 * (z + 0.044715 * z * z * z)))


def kernel(A, B):
    m, k_per = A.shape
    _, n = B.shape

    def body(
        a_hbm, b_hbm, out_hbm,
        a_ref, b_ref, acc_ref, b16_ref,
        sbuf_p, rbuf_p, sbuf_m, rbuf_m,
        send_p, recv_p, send_m, recv_m,
        credit_p, credit_m,
        in_sems, out_sems,
    ):
        cp_a = pltpu.make_async_copy(a_hbm, a_ref, in_sems.at[0])
        cp_b = pltpu.make_async_copy(b_hbm, b_ref, in_sems.at[1])
        cp_a.start()
        cp_b.start()

        my_log = lax.axis_index("i")
        my = jnp.int32(0)
        right = jnp.int32(0)
        left = jnp.int32(0)
        for j in range(N_DEV):
            here = my_log == RING[j]
            my = jnp.where(here, j, my)
            right = jnp.where(here, RING[(j + 1) % N_DEV], right)
            left = jnp.where(here, RING[(j - 1) % N_DEV], left)

        barrier_sem = pltpu.get_barrier_semaphore()
        for nbr in (left, right):
            pl.semaphore_signal(
                barrier_sem, inc=1, device_id=(nbr,),
                device_id_type=pl.DeviceIdType.MESH,
            )
        pl.semaphore_wait(barrier_sem, 2)

        cp_b.wait()
        b16_ref[...] = b_ref[...].astype(jnp.bfloat16)
        cp_a.wait()

        def tq(c, q):
            return pl.ds(c * CHUNK + q * QROWS, QROWS)

        def bq(c, q):
            return pl.ds(c * CHUNK + HALF + q * QROWS, QROWS)

        def dot_half(r0):
            acc_ref[pl.ds(r0, HALF), :] = jnp.dot(
                a_ref[pl.ds(r0, HALF), :].astype(jnp.bfloat16), b16_ref[...],
                preferred_element_type=jnp.float32,
            )

        sems = {
            "p": (send_p, recv_p, credit_p, sbuf_p, rbuf_p, right, left),
            "m": (send_m, recv_m, credit_m, sbuf_m, rbuf_m, left, right),
        }
        pend = {"p": {}, "m": {}}

        def issue(d, mm, payload):
            send_sems, recv_sems, credit, sbuf, rbuf, tgt, _ = sems[d]
            slot = mm % NSLOT
            if mm >= NSLOT:
                pend[d][mm - NSLOT].wait_send()
                pl.semaphore_wait(credit, 1)
            sbuf[slot] = payload
            desc = pltpu.make_async_remote_copy(
                src_ref=sbuf.at[slot],
                dst_ref=rbuf.at[slot],
                send_sem=send_sems.at[slot],
                recv_sem=recv_sems.at[slot],
                device_id=(tgt,),
                device_id_type=pl.DeviceIdType.MESH,
            )
            desc.start()
            pend[d][mm] = desc

        def credit_back(d, mm):
            if mm < NMSG - NSLOT:
                _s, _r, credit, _sb, _rb, _tgt, src_nbr = sems[d]
                pl.semaphore_signal(
                    credit, inc=1, device_id=(src_nbr,),
                    device_id_type=pl.DeviceIdType.MESH,
                )

        on = [0]
        opend = {}

        def store_out(row_slice):
            i = on[0]
            on[0] += 1
            slot = i % OSLOT
            if i >= OSLOT:
                opend[i - OSLOT].wait()
            c = pltpu.make_async_copy(
                acc_ref.at[row_slice, :], out_hbm.at[row_slice, :],
                out_sems.at[slot],
            )
            c.start()
            opend[i] = c

        dot_half(my * CHUNK)
        dot_half(my * CHUNK + HALF)
        for q in range(Q):
            issue("p", q, acc_ref[tq(my, q), :].astype(jnp.bfloat16))
            issue("m", q, acc_ref[bq(my, q), :].astype(jnp.bfloat16))

        for s in range(RS):
            cp = jnp.mod(my - s - 1, N_DEV)
            cm = jnp.mod(my + s + 1, N_DEV)
            dot_half(cp * CHUNK)
            dot_half(cm * CHUNK + HALF)
            for q in range(Q):
                mm = s * Q + q
                pend["p"][mm].wait_recv()
                if s < RS - 1:
                    issue(
                        "p", mm + Q,
                        (
                            acc_ref[tq(cp, q), :]
                            + rbuf_p[mm % NSLOT].astype(jnp.float32)
                        ).astype(jnp.bfloat16),
                    )
                else:
                    acc_ref[tq(cp, q), :] = _gelu(
                        acc_ref[tq(cp, q), :]
                        + rbuf_p[mm % NSLOT].astype(jnp.float32)
                    )
                    issue("p", RS * Q + q, acc_ref[tq(cp, q), :].astype(jnp.bfloat16))
                credit_back("p", mm)
                pend["m"][mm].wait_recv()
                if s < RS - 1:
                    issue(
                        "m", mm + Q,
                        (
                            acc_ref[bq(cm, q), :]
                            + rbuf_m[mm % NSLOT].astype(jnp.float32)
                        ).astype(jnp.bfloat16),
                    )
                else:
                    acc_ref[bq(cm, q), :] = _gelu(
                        acc_ref[bq(cm, q), :]
                        + rbuf_m[mm % NSLOT].astype(jnp.float32)
                    )
                    issue("m", RS * Q + q, acc_ref[bq(cm, q), :].astype(jnp.bfloat16))
                credit_back("m", mm)

        own_p = jnp.mod(my + 1, N_DEV)
        own_m = jnp.mod(my - 1, N_DEV)
        store_out(pl.ds(own_p * CHUNK, HALF))
        store_out(pl.ds(own_m * CHUNK + HALF, HALF))

        for t in range(RS):
            rp = jnp.mod(my - t, N_DEV)
            rm = jnp.mod(my + t, N_DEV)
            for q in range(Q):
                mm = (RS + t) * Q + q
                pend["p"][mm].wait_recv()
                if t < RS - 1:
                    issue("p", mm + Q, rbuf_p[mm % NSLOT])
                acc_ref[tq(rp, q), :] = rbuf_p[mm % NSLOT].astype(jnp.float32)
                credit_back("p", mm)
                store_out(tq(rp, q))
                pend["m"][mm].wait_recv()
                if t < RS - 1:
                    issue("m", mm + Q, rbuf_m[mm % NSLOT])
                acc_ref[bq(rm, q), :] = rbuf_m[mm % NSLOT].astype(jnp.float32)
                credit_back("m", mm)
                store_out(bq(rm, q))

        for mm in range(NMSG - NSLOT, NMSG):
            pend["p"][mm].wait_send()
            pend["m"][mm].wait_send()
        for i in range(max(0, on[0] - OSLOT), on[0]):
            opend[i].wait()

        @functools.partial(pl.run_scoped, sem=pltpu.SemaphoreType.REGULAR)
        def _(sem):
            for nbr in (left, right):
                pl.semaphore_signal(
                    sem, inc=1, device_id=(nbr,),
                    device_id_type=pl.DeviceIdType.MESH,
                )
            pl.semaphore_wait(sem, 2)

    return pl.pallas_call(
        body,
        out_shape=jax.ShapeDtypeStruct((m, n), jnp.float32),
        in_specs=[
            pl.BlockSpec(memory_space=pl.ANY),
            pl.BlockSpec(memory_space=pl.ANY),
        ],
        out_specs=pl.BlockSpec(memory_space=pl.ANY),
        scratch_shapes=[
            pltpu.VMEM((M, k_per), jnp.float32),
            pltpu.VMEM((k_per, N), jnp.float32),
            pltpu.VMEM((M, N), jnp.float32),
            pltpu.VMEM((k_per, N), jnp.bfloat16),
            pltpu.VMEM((NSLOT, QROWS, N), jnp.bfloat16),
            pltpu.VMEM((NSLOT, QROWS, N), jnp.bfloat16),
            pltpu.VMEM((NSLOT, QROWS, N), jnp.bfloat16),
            pltpu.VMEM((NSLOT, QROWS, N), jnp.bfloat16),
            pltpu.SemaphoreType.DMA((NSLOT,)),
            pltpu.SemaphoreType.DMA((NSLOT,)),
            pltpu.SemaphoreType.DMA((NSLOT,)),
            pltpu.SemaphoreType.DMA((NSLOT,)),
            pltpu.SemaphoreType.REGULAR,
            pltpu.SemaphoreType.REGULAR,
            pltpu.SemaphoreType.DMA((2,)),
            pltpu.SemaphoreType.DMA((OSLOT,)),
        ],
        compiler_params=pltpu.CompilerParams(
            collective_id=0,
            vmem_limit_bytes=100 * 1024 * 1024,
        ),
    )(A, B)
